# baseline (speedup 1.0000x reference)
import jax
import jax.numpy as jnp
from jax.experimental import pallas as pl
from jax.experimental.pallas import tpu as pltpu

_CC = 16


def _sem_kernel(fmapd_ref, W1_ref, word_ref, W2_ref, W3_ref, Wa_ref,
                out_ref, fd_ref, vblk_ref):
    C = word_ref.shape[0]
    M = W2_ref.shape[0]

    @pl.when(pl.program_id(0) == 0)
    def _prep():
        f_wd = jax.lax.dot_general(
            word_ref[...], W2_ref[...], (((1,), (1,)), ((), ())),
            preferred_element_type=jnp.float32)
        for c in range(C):
            fd_ref[0:1, c * M:(c + 1) * M] = f_wd[c:c + 1, :]
        vT = jax.lax.dot_general(
            W3_ref[...], Wa_ref[...], (((0,), (1,)), ((), ())),
            preferred_element_type=jnp.float32)
        vb = jnp.broadcast_to(vT, (M, _CC))
        lane = jax.lax.broadcasted_iota(jnp.int32, (M, _CC), 1)
        for j in range(_CC):
            vblk_ref[j * M:(j + 1) * M, :] = jnp.where(
                lane == j, vb, 0.0).astype(jnp.bfloat16)

    fmap_d = fmapd_ref[0]
    f_wh = jax.lax.dot_general(
        fmap_d, W1_ref[...], (((0,), (1,)), ((), ())),
        preferred_element_type=jnp.float32)
    fwh_rep = pltpu.repeat(f_wh, _CC, axis=1)
    vblk = vblk_ref[...]

    cols = []
    for j in range(0, C, _CC):
        fd_j = fd_ref[...][:, j * M:(j + _CC) * M]
        t = jnp.tanh(fwh_rep * fd_j)
        cols.append(jax.lax.dot_general(
            t.astype(jnp.bfloat16), vblk, (((1,), (0,)), ((), ())),
            preferred_element_type=jnp.float32))
    coef = jnp.concatenate(cols, axis=1)

    coef = coef - jnp.max(coef, axis=0, keepdims=True)
    e = jnp.exp(coef)
    coef = e / jnp.sum(e, axis=0, keepdims=True)

    out_ref[0] = jax.lax.dot_general(
        coef, fmap_d, (((0,), (1,)), ((), ())),
        preferred_element_type=jnp.float32)


def kernel(batch_size, img_feature_map, word_features, W1, W2, W3, b3, Wa, ba):
    Bn, D, H, W = img_feature_map.shape
    S = H * W
    fmap_d = img_feature_map.reshape(Bn, D, S)
    C, DW = word_features.shape
    M = W1.shape[0]
    return pl.pallas_call(
        _sem_kernel,
        grid=(Bn,),
        in_specs=[
            pl.BlockSpec((1, D, S), lambda b: (b, 0, 0)),
            pl.BlockSpec((M, D), lambda b: (0, 0)),
            pl.BlockSpec((C, DW), lambda b: (0, 0)),
            pl.BlockSpec((M, DW), lambda b: (0, 0)),
            pl.BlockSpec((M, M), lambda b: (0, 0)),
            pl.BlockSpec((1, M), lambda b: (0, 0)),
        ],
        out_specs=pl.BlockSpec((1, C, D), lambda b: (b, 0, 0)),
        out_shape=jax.ShapeDtypeStruct((Bn, C, D), jnp.float32),
        scratch_shapes=[
            pltpu.VMEM((1, C * M), jnp.float32),
            pltpu.VMEM((_CC * M, _CC), jnp.bfloat16),
        ],
        compiler_params=pltpu.CompilerParams(
            dimension_semantics=("arbitrary",),
            vmem_limit_bytes=56 * 1024 * 1024,
        ),
    )(fmap_d, W1, word_features, W2, W3, Wa)


# speedup vs baseline: 1.1208x; 1.1208x over previous
import jax
import jax.numpy as jnp
from jax.experimental import pallas as pl
from jax.experimental.pallas import tpu as pltpu

_CC = 16


def _sem_kernel(fmapd_ref, W1_ref, word_ref, W2_ref, W3_ref, Wa_ref,
                out_ref, fd_ref, v_ref):
    C = word_ref.shape[0]
    M = W2_ref.shape[0]

    @pl.when(pl.program_id(0) == 0)
    def _prep():
        f_wd = jax.lax.dot_general(
            word_ref[...], W2_ref[...], (((1,), (1,)), ((), ())),
            preferred_element_type=jnp.float32)
        for c in range(C):
            fd_ref[0:1, c * M:(c + 1) * M] = f_wd[c:c + 1, :]
        v_ref[...] = jax.lax.dot_general(
            Wa_ref[...], W3_ref[...], (((1,), (0,)), ((), ())),
            preferred_element_type=jnp.float32)

    fmap_d = fmapd_ref[0]
    f_wh = jax.lax.dot_general(
        fmap_d, W1_ref[...], (((0,), (1,)), ((), ())),
        preferred_element_type=jnp.float32)
    fwh_rep = pltpu.repeat(f_wh, _CC, axis=1)
    vrow = v_ref[...]

    cols = []
    for j in range(0, C, _CC):
        fd_j = fd_ref[...][:, j * M:(j + _CC) * M]
        t = jnp.tanh(fwh_rep * fd_j)
        for k in range(_CC):
            w = t[:, k * M:(k + 1) * M] * vrow
            cols.append(jnp.sum(w, axis=1, keepdims=True))
    coef = jnp.concatenate(cols, axis=1)

    coef = coef - jnp.max(coef, axis=0, keepdims=True)
    e = jnp.exp(coef)
    coef = e / jnp.sum(e, axis=0, keepdims=True)

    out_ref[0] = jax.lax.dot_general(
        coef, fmap_d, (((0,), (1,)), ((), ())),
        preferred_element_type=jnp.float32)


def kernel(batch_size, img_feature_map, word_features, W1, W2, W3, b3, Wa, ba):
    Bn, D, H, W = img_feature_map.shape
    S = H * W
    fmap_d = img_feature_map.reshape(Bn, D, S)
    C, DW = word_features.shape
    M = W1.shape[0]
    return pl.pallas_call(
        _sem_kernel,
        grid=(Bn,),
        in_specs=[
            pl.BlockSpec((1, D, S), lambda b: (b, 0, 0)),
            pl.BlockSpec((M, D), lambda b: (0, 0)),
            pl.BlockSpec((C, DW), lambda b: (0, 0)),
            pl.BlockSpec((M, DW), lambda b: (0, 0)),
            pl.BlockSpec((M, M), lambda b: (0, 0)),
            pl.BlockSpec((1, M), lambda b: (0, 0)),
        ],
        out_specs=pl.BlockSpec((1, C, D), lambda b: (b, 0, 0)),
        out_shape=jax.ShapeDtypeStruct((Bn, C, D), jnp.float32),
        scratch_shapes=[
            pltpu.VMEM((1, C * M), jnp.float32),
            pltpu.VMEM((1, M), jnp.float32),
        ],
        compiler_params=pltpu.CompilerParams(
            dimension_semantics=("arbitrary",),
            vmem_limit_bytes=56 * 1024 * 1024,
        ),
    )(fmap_d, W1, word_features, W2, W3, Wa)
